# baseline (speedup 1.0000x reference)
import functools

import jax
import jax.numpy as jnp
from jax import lax
from jax.experimental import pallas as pl
from jax.experimental.pallas import tpu as pltpu
from jax.experimental.pallas import tpu_sc as plsc


def _sc_gather(weight, ids):
    info = plsc.get_sparse_core_info()
    nc, ns = info.num_cores, info.num_subcores
    nw = nc * ns
    n = ids.shape[0]
    d = weight.shape[1]
    b_per_w = n // nw
    mesh = plsc.VectorSubcoreMesh(core_axis_name="c", subcore_axis_name="s")

    @functools.partial(
        pl.kernel,
        mesh=mesh,
        out_type=jax.ShapeDtypeStruct((n, d), jnp.float32),
        scratch_types=[
            pltpu.VMEM((b_per_w,), jnp.int32),
            pltpu.VMEM((b_per_w, d), jnp.float32),
            pltpu.SemaphoreType.DMA,
        ],
    )
    def gather_kernel(table_hbm, idx_hbm, out_hbm, idx_v, rows_v, sem):
        wid = lax.axis_index("s") * nc + lax.axis_index("c")
        base = wid * b_per_w
        pltpu.sync_copy(idx_hbm.at[pl.ds(base, b_per_w)], idx_v)
        pltpu.async_copy(table_hbm.at[idx_v], rows_v, sem).wait()
        pltpu.sync_copy(rows_v, out_hbm.at[pl.ds(base, b_per_w)])

    return gather_kernel(weight, ids)


def _matmul_body(x_ref, w_ref, o_ref):
    o_ref[...] = lax.dot_general(
        x_ref[...], w_ref[...],
        dimension_numbers=(((1,), (1,)), ((), ())),
        preferred_element_type=jnp.float32,
    )


def _tc_logits(x, weight, tv):
    n, e = x.shape
    v = weight.shape[0]
    grid = pl.cdiv(v, tv)
    return pl.pallas_call(
        _matmul_body,
        grid=(grid,),
        in_specs=[
            pl.BlockSpec((n, e), lambda i: (0, 0)),
            pl.BlockSpec((tv, e), lambda i: (i, 0)),
        ],
        out_specs=pl.BlockSpec((n, tv), lambda i: (0, i)),
        out_shape=jax.ShapeDtypeStruct((n, v), jnp.float32),
    )(x, weight)


def kernel(input_ids, weight):
    b, s = input_ids.shape
    v, e = weight.shape
    n = b * s
    ids = input_ids.reshape(n)
    x = _sc_gather(weight, ids)
    logits = _tc_logits(x, weight, tv=2048)
    return logits.reshape(b, s, v)


# speedup vs baseline: 1.0043x; 1.0043x over previous
import functools

import jax
import jax.numpy as jnp
from jax import lax
from jax.experimental import pallas as pl
from jax.experimental.pallas import tpu as pltpu
from jax.experimental.pallas import tpu_sc as plsc


def _sc_gather(weight, ids):
    info = plsc.get_sparse_core_info()
    nc, ns = info.num_cores, info.num_subcores
    nw = nc * ns
    n = ids.shape[0]
    d = weight.shape[1]
    b_per_w = n // nw
    mesh = plsc.VectorSubcoreMesh(core_axis_name="c", subcore_axis_name="s")

    @functools.partial(
        pl.kernel,
        mesh=mesh,
        out_type=jax.ShapeDtypeStruct((n, d), jnp.float32),
        scratch_types=[
            pltpu.VMEM((b_per_w,), jnp.int32),
            pltpu.VMEM((b_per_w, d), jnp.float32),
            pltpu.SemaphoreType.DMA,
        ],
    )
    def gather_kernel(table_hbm, idx_hbm, out_hbm, idx_v, rows_v, sem):
        wid = lax.axis_index("s") * nc + lax.axis_index("c")
        base = wid * b_per_w
        pltpu.sync_copy(idx_hbm.at[pl.ds(base, b_per_w)], idx_v)
        pltpu.async_copy(table_hbm.at[idx_v], rows_v, sem).wait()
        pltpu.sync_copy(rows_v, out_hbm.at[pl.ds(base, b_per_w)])

    return gather_kernel(weight, ids)


def _matmul_body(x_ref, w_ref, o_ref):
    o_ref[...] = lax.dot_general(
        x_ref[...].astype(jnp.bfloat16), w_ref[...].astype(jnp.bfloat16),
        dimension_numbers=(((1,), (1,)), ((), ())),
        preferred_element_type=jnp.float32,
    )


def _tc_logits(x, weight, tv):
    n, e = x.shape
    v = weight.shape[0]
    grid = pl.cdiv(v, tv)
    return pl.pallas_call(
        _matmul_body,
        grid=(grid,),
        in_specs=[
            pl.BlockSpec((n, e), lambda i: (0, 0)),
            pl.BlockSpec((tv, e), lambda i: (i, 0)),
        ],
        out_specs=pl.BlockSpec((n, tv), lambda i: (0, i)),
        out_shape=jax.ShapeDtypeStruct((n, v), jnp.float32),
    )(x, weight)


def kernel(input_ids, weight):
    b, s = input_ids.shape
    v, e = weight.shape
    n = b * s
    ids = input_ids.reshape(n)
    x = _sc_gather(weight, ids)
    logits = _tc_logits(x, weight, tv=2048)
    return logits.reshape(b, s, v)


# speedup vs baseline: 1.0208x; 1.0165x over previous
import functools

import jax
import jax.numpy as jnp
from jax import lax
from jax.experimental import pallas as pl
from jax.experimental.pallas import tpu as pltpu
from jax.experimental.pallas import tpu_sc as plsc


def _sc_gather(weight, ids):
    info = plsc.get_sparse_core_info()
    nc, ns = info.num_cores, info.num_subcores
    nw = nc * ns
    n = ids.shape[0]
    d = weight.shape[1]
    b_per_w = n // nw
    mesh = plsc.VectorSubcoreMesh(core_axis_name="c", subcore_axis_name="s")

    @functools.partial(
        pl.kernel,
        mesh=mesh,
        out_type=jax.ShapeDtypeStruct((n, d), jnp.float32),
        scratch_types=[
            pltpu.VMEM((b_per_w,), jnp.int32),
            pltpu.VMEM((b_per_w, d), jnp.float32),
            pltpu.SemaphoreType.DMA,
        ],
    )
    def gather_kernel(table_hbm, idx_hbm, out_hbm, idx_v, rows_v, sem):
        wid = lax.axis_index("s") * nc + lax.axis_index("c")
        base = wid * b_per_w
        pltpu.sync_copy(idx_hbm.at[pl.ds(base, b_per_w)], idx_v)
        pltpu.async_copy(table_hbm.at[idx_v], rows_v, sem).wait()
        pltpu.sync_copy(rows_v, out_hbm.at[pl.ds(base, b_per_w)])

    return gather_kernel(weight, ids)


def _matmul_body(x_ref, w_ref, o_ref):
    o_ref[...] = lax.dot_general(
        x_ref[...].astype(jnp.bfloat16), w_ref[...].astype(jnp.bfloat16),
        dimension_numbers=(((1,), (1,)), ((), ())),
        preferred_element_type=jnp.float32,
    )


def _tc_logits(x, weight, tv):
    n, e = x.shape
    v = weight.shape[0]
    grid = pl.cdiv(v, tv)
    return pl.pallas_call(
        _matmul_body,
        grid=(grid,),
        in_specs=[
            pl.BlockSpec((n, e), lambda i: (0, 0)),
            pl.BlockSpec((tv, e), lambda i: (i, 0)),
        ],
        out_specs=pl.BlockSpec((n, tv), lambda i: (0, i)),
        out_shape=jax.ShapeDtypeStruct((n, v), jnp.float32),
    )(x, weight)


def kernel(input_ids, weight):
    b, s = input_ids.shape
    v, e = weight.shape
    n = b * s
    ids = input_ids.reshape(n)
    x = _sc_gather(weight, ids)
    logits = _tc_logits(x, weight, tv=4096)
    return logits.reshape(b, s, v)


# speedup vs baseline: 1.0245x; 1.0036x over previous
import functools

import jax
import jax.numpy as jnp
from jax import lax
from jax.experimental import pallas as pl
from jax.experimental.pallas import tpu as pltpu
from jax.experimental.pallas import tpu_sc as plsc


def _sc_gather(weight, ids):
    info = plsc.get_sparse_core_info()
    nc, ns = info.num_cores, info.num_subcores
    nw = nc * ns
    n = ids.shape[0]
    d = weight.shape[1]
    b_per_w = n // nw
    mesh = plsc.VectorSubcoreMesh(core_axis_name="c", subcore_axis_name="s")

    @functools.partial(
        pl.kernel,
        mesh=mesh,
        out_type=jax.ShapeDtypeStruct((n, d), jnp.float32),
        scratch_types=[
            pltpu.VMEM((b_per_w,), jnp.int32),
            pltpu.VMEM((b_per_w, d), jnp.float32),
            pltpu.SemaphoreType.DMA,
        ],
    )
    def gather_kernel(table_hbm, idx_hbm, out_hbm, idx_v, rows_v, sem):
        wid = lax.axis_index("s") * nc + lax.axis_index("c")
        base = wid * b_per_w
        pltpu.sync_copy(idx_hbm.at[pl.ds(base, b_per_w)], idx_v)
        pltpu.async_copy(table_hbm.at[idx_v], rows_v, sem).wait()
        pltpu.sync_copy(rows_v, out_hbm.at[pl.ds(base, b_per_w)])

    return gather_kernel(weight, ids)


def _matmul_body(x_ref, w_ref, o_ref):
    o_ref[...] = lax.dot_general(
        x_ref[...].astype(jnp.bfloat16), w_ref[...].astype(jnp.bfloat16),
        dimension_numbers=(((1,), (1,)), ((), ())),
        preferred_element_type=jnp.float32,
    )


def _tc_logits(x, weight, tv):
    n, e = x.shape
    v = weight.shape[0]
    grid = pl.cdiv(v, tv)
    return pl.pallas_call(
        _matmul_body,
        grid=(grid,),
        in_specs=[
            pl.BlockSpec((n, e), lambda i: (0, 0)),
            pl.BlockSpec((tv, e), lambda i: (i, 0)),
        ],
        out_specs=pl.BlockSpec((n, tv), lambda i: (0, i)),
        out_shape=jax.ShapeDtypeStruct((n, v), jnp.float32),
    )(x, weight)


def kernel(input_ids, weight):
    b, s = input_ids.shape
    v, e = weight.shape
    n = b * s
    ids = input_ids.reshape(n)
    x = _sc_gather(weight, ids)
    logits = _tc_logits(x, weight, tv=5120)
    return logits.reshape(b, s, v)
